# baseline (speedup 1.0000x reference)
import jax
import jax.numpy as jnp
from jax.experimental import pallas as pl
from jax.experimental.pallas import tpu as pltpu

_NUM_CLASSES = 30522
_ROWS = 1024
_BC = 1536


def _onehot_block(x_ref, t_ref):
    i = pl.program_id(0)
    classes = jax.lax.broadcasted_iota(jnp.int32, (_BC, _ROWS), 0) + i * _BC
    t_ref[...] = (x_ref[...] == classes).astype(jnp.int32)


def kernel(x):
    x2 = x.reshape(1, _ROWS)
    t = pl.pallas_call(
        _onehot_block,
        grid=(pl.cdiv(_NUM_CLASSES, _BC),),
        in_specs=[pl.BlockSpec((1, _ROWS), lambda i: (0, 0))],
        out_specs=pl.BlockSpec((_BC, _ROWS), lambda i: (i, 0)),
        out_shape=jax.ShapeDtypeStruct((_NUM_CLASSES, _ROWS), jnp.int32),
        compiler_params=pltpu.CompilerParams(
            dimension_semantics=("parallel",),
        ),
    )(x2)
    return t.T


# speedup vs baseline: 1.0062x; 1.0062x over previous
import jax
import jax.numpy as jnp
from jax.experimental import pallas as pl
from jax.experimental.pallas import tpu as pltpu

_NUM_CLASSES = 30522
_ROWS = 1024
_BC = 768


def _onehot_block(x_ref, t_ref):
    i = pl.program_id(0)
    classes = jax.lax.broadcasted_iota(jnp.int32, (_BC, _ROWS), 0) + i * _BC
    t_ref[...] = (x_ref[...] == classes).astype(jnp.int32)


def kernel(x):
    x2 = x.reshape(1, _ROWS)
    t = pl.pallas_call(
        _onehot_block,
        grid=(pl.cdiv(_NUM_CLASSES, _BC),),
        in_specs=[pl.BlockSpec((1, _ROWS), lambda i: (0, 0))],
        out_specs=pl.BlockSpec((_BC, _ROWS), lambda i: (i, 0)),
        out_shape=jax.ShapeDtypeStruct((_NUM_CLASSES, _ROWS), jnp.int32),
        compiler_params=pltpu.CompilerParams(
            dimension_semantics=("parallel",),
        ),
    )(x2)
    return t.T


# speedup vs baseline: 1.0425x; 1.0361x over previous
import jax
import jax.numpy as jnp
from jax.experimental import pallas as pl
from jax.experimental.pallas import tpu as pltpu

_NUM_CLASSES = 30522
_ROWS = 1024
_BC = 1024


def _onehot_block(x_ref, t_ref):
    i = pl.program_id(0)
    classes = jax.lax.broadcasted_iota(jnp.int32, (_BC, _ROWS), 0) + i * _BC
    t_ref[...] = (x_ref[...] == classes).astype(jnp.int32)


def kernel(x):
    x2 = x.reshape(1, _ROWS)
    t = pl.pallas_call(
        _onehot_block,
        grid=(pl.cdiv(_NUM_CLASSES, _BC),),
        in_specs=[pl.BlockSpec((1, _ROWS), lambda i: (0, 0))],
        out_specs=pl.BlockSpec((_BC, _ROWS), lambda i: (i, 0)),
        out_shape=jax.ShapeDtypeStruct((_NUM_CLASSES, _ROWS), jnp.int32),
        compiler_params=pltpu.CompilerParams(
            dimension_semantics=("parallel",),
        ),
    )(x2)
    return t.T
